# Initial kernel scaffold; baseline (speedup 1.0000x reference)
import jax
import jax.numpy as jnp
from jax.experimental import pallas as pl


def kernel(x, edge_index, instr_vectors, batch, Ws, bs, gammas, betas):
    raise NotImplementedError("write your pallas kernel here")


# speedup vs baseline: 1.6556x; 1.6556x over previous
import jax
import jax.numpy as jnp
from jax.experimental import pallas as pl

_EPS = 1e-5


def _bn_relu_chain_kernel(x_ref, g_ref, b_ref, o_ref):
    h = x_ref[...]
    n = jnp.float32(x_ref.shape[0])
    for i in range(g_ref.shape[0]):
        s1 = jnp.sum(h, axis=0, keepdims=True)
        s2 = jnp.sum(h * h, axis=0, keepdims=True)
        mean = s1 / n
        var = s2 / n - mean * mean
        scale = jax.lax.rsqrt(var + _EPS) * g_ref[i][None, :]
        shift = b_ref[i][None, :] - mean * scale
        h = jnp.maximum(h * scale + shift, 0.0)
    o_ref[...] = h


def kernel(x, edge_index, instr_vectors, batch, Ws, bs, gammas, betas):
    del edge_index, instr_vectors, batch, Ws, bs
    return pl.pallas_call(
        _bn_relu_chain_kernel,
        out_shape=jax.ShapeDtypeStruct(x.shape, x.dtype),
    )(x, gammas, betas)
